# baseline (speedup 1.0000x reference)
import functools
import math

import jax
import jax.numpy as jnp
from jax import lax
from jax.experimental import pallas as pl
from jax.experimental.pallas import tpu as pltpu
from jax.experimental.pallas import tpu_sc as plsc

_D = 64
_NC, _NS = 2, 16
_NW = _NC * _NS
_LANES = 16
_SCALE = math.sqrt(_D)
_NBUF = 4


@jax.jit
def _embed_lookup(tokens, table):
    nrows, hist = tokens.shape
    rows_per_w = nrows // _NW
    mesh = plsc.VectorSubcoreMesh(
        core_axis_name="c", subcore_axis_name="s",
        num_cores=_NC, num_subcores=_NS)

    @functools.partial(
        pl.kernel,
        out_type=jax.ShapeDtypeStruct((nrows, hist, _D), jnp.float32),
        mesh=mesh,
        compiler_params=pltpu.CompilerParams(use_tc_tiling_on_sc=False),
        scratch_types=[
            pltpu.VMEM((_NBUF, hist), jnp.int32),
            pltpu.VMEM((_NBUF, hist, _D), jnp.float32),
        ] + [pltpu.SemaphoreType.DMA] * (2 * _NBUF),
    )
    def k(tokens_hbm, table_hbm, out_hbm, idx_v, rows_v, *sems):
        gsems = sems[:_NBUF]
        osems = sems[_NBUF:]
        wid = lax.axis_index("s") * _NC + lax.axis_index("c")
        base = wid * rows_per_w

        def start_gather(buf, row, gsem):
            pltpu.sync_copy(tokens_hbm.at[base + row], idx_v.at[buf])
            pltpu.async_copy(table_hbm.at[idx_v.at[buf]], rows_v.at[buf], gsem)

        for b in range(_NBUF - 1):
            start_gather(b, b, gsems[b])

        @pl.loop(0, rows_per_w, step=_NBUF)
        def _outer(G):
            for b in range(_NBUF):
                g = G + b
                nb = (b + _NBUF - 1) % _NBUF

                @pl.when(g + _NBUF - 1 < rows_per_w)
                def _start_next():
                    @pl.when(g >= 1)
                    def _drain():
                        pltpu.make_async_copy(
                            rows_v.at[nb], out_hbm.at[base], osems[nb]).wait()
                    start_gather(nb, g + _NBUF - 1, gsems[nb])

                pltpu.make_async_copy(
                    table_hbm.at[idx_v.at[b]], rows_v.at[b], gsems[b]).wait()

                @pl.loop(0, hist, unroll=8)
                def _scale(r):
                    for j in range(_D // _LANES):
                        sl = pl.ds(j * _LANES, _LANES)
                        rows_v[b, r, sl] = rows_v[b, r, sl] * _SCALE

                pltpu.async_copy(rows_v.at[b], out_hbm.at[base + g], osems[b])

        for b in range(_NBUF):
            pltpu.make_async_copy(
                rows_v.at[b], out_hbm.at[base], osems[b]).wait()

    return k(tokens, table)


def kernel(tokens, table):
    return _embed_lookup(tokens.astype(jnp.int32), table)
